# baseline (speedup 1.0000x reference)
import functools

import jax
import jax.numpy as jnp
from jax import lax
from jax.experimental import pallas as pl
from jax.experimental.pallas import tpu as pltpu
from jax.experimental.pallas import tpu_sc as plsc

_NUM_CORES = 2
_NUM_SUBCORES = 16
_NUM_WORKERS = _NUM_CORES * _NUM_SUBCORES


def _gather_body(b_per_w, table_hbm, idx_hbm, out_hbm, idx_v, rows_v, sem):
    wid = lax.axis_index("s") * _NUM_CORES + lax.axis_index("c")
    base = wid * b_per_w
    pltpu.sync_copy(idx_hbm.at[pl.ds(base, b_per_w)], idx_v)
    pltpu.async_copy(table_hbm.at[idx_v], rows_v, sem).wait()
    pltpu.sync_copy(rows_v, out_hbm.at[pl.ds(base, b_per_w)])


def _sc_gather(table, idx_flat):
    n_idx = idx_flat.shape[0]
    embed = table.shape[1]
    b_per_w = n_idx // _NUM_WORKERS
    mesh = plsc.VectorSubcoreMesh(core_axis_name="c", subcore_axis_name="s")
    kern = pl.kernel(
        functools.partial(_gather_body, b_per_w),
        mesh=mesh,
        out_type=jax.ShapeDtypeStruct((n_idx, embed), jnp.float32),
        scratch_types=[
            pltpu.VMEM((b_per_w,), jnp.int32),
            pltpu.VMEM((b_per_w, embed), jnp.float32),
            pltpu.SemaphoreType.DMA,
        ],
        compiler_params=pltpu.CompilerParams(use_tc_tiling_on_sc=False),
    )
    return kern(table, idx_flat)


def _mm_body_t(bt_ref, at_ref, ot_ref):
    ot_ref[...] = jnp.dot(
        bt_ref[...], at_ref[...], preferred_element_type=jnp.float32
    )


def _tc_matmul_t(at, bt, col_block):
    k, m = at.shape
    n, _ = bt.shape
    return pl.pallas_call(
        _mm_body_t,
        grid=(m // col_block,),
        in_specs=[
            pl.BlockSpec((n, k), lambda i: (0, 0)),
            pl.BlockSpec((k, col_block), lambda i: (0, i)),
        ],
        out_specs=pl.BlockSpec((n, col_block), lambda i: (0, i)),
        out_shape=jax.ShapeDtypeStruct((n, m), jnp.float32),
    )(bt, at)


def kernel(node_idx, node_val_mat, node_embed_table, val_tok_embed):
    l, n, b = node_idx.shape
    e = node_embed_table.shape[1]
    idx_flat = node_idx.reshape(-1)
    node_embed = _sc_gather(node_embed_table, idx_flat).reshape(l, n, b, e)
    out_t = _tc_matmul_t(node_val_mat.T, val_tok_embed.T, 3200)
    node_val_embed = out_t.T.reshape(l, n, b, e)
    return node_embed, node_val_embed
